# baseline (speedup 1.0000x reference)
import functools

import jax
import jax.numpy as jnp
from jax.experimental import pallas as pl
from jax.experimental.pallas import tpu as pltpu

_NBUF = 4
_BLK = 256


def _smoothness_body(a_hbm, z_ref, out_ref, buf, sem, *, inv_n, nchunks):
    zfull = z_ref[...]

    def start(k, slot):
        pltpu.make_async_copy(
            a_hbm.at[pl.ds(k * _BLK, _BLK), :], buf.at[slot], sem.at[slot]
        ).start()

    for k in range(_NBUF):
        start(k, k)

    def step(k, acc):
        slot = jax.lax.rem(k, _NBUF)
        pltpu.make_async_copy(
            a_hbm.at[pl.ds(k * _BLK, _BLK), :], buf.at[slot], sem.at[slot]
        ).wait()
        a = buf[slot]
        zi = z_ref[pl.ds(k * _BLK, _BLK), :]
        y = jnp.dot(a, zfull, preferred_element_type=jnp.float32)
        d = jnp.sum(a, axis=1)
        s = jnp.sum(zi * zi, axis=1)
        acc += jnp.sum(d * s) - jnp.sum(zi * y)

        @pl.when(k + _NBUF < nchunks)
        def _():
            start(k + _NBUF, slot)

        return acc

    acc = jax.lax.fori_loop(0, nchunks, step, jnp.float32(0.0))
    out_ref[...] = jnp.reshape(acc * inv_n, (1, 1))


@jax.jit
def kernel(z, coords, precomputed_adj):
    del coords
    n, dim = z.shape
    nchunks = n // _BLK
    out = pl.pallas_call(
        functools.partial(_smoothness_body, inv_n=1.0 / n, nchunks=nchunks),
        in_specs=[
            pl.BlockSpec(memory_space=pltpu.MemorySpace.HBM),
            pl.BlockSpec(memory_space=pltpu.MemorySpace.VMEM),
        ],
        out_specs=pl.BlockSpec(memory_space=pltpu.MemorySpace.VMEM),
        out_shape=jax.ShapeDtypeStruct((1, 1), jnp.float32),
        scratch_shapes=[
            pltpu.VMEM((_NBUF, _BLK, 4096), jnp.float32),
            pltpu.SemaphoreType.DMA((_NBUF,)),
        ],
    )(precomputed_adj, z)
    return out[0, 0]


# speedup vs baseline: 1.0349x; 1.0349x over previous
import functools

import jax
import jax.numpy as jnp
from jax.experimental import pallas as pl


def _smoothness_body(a0_ref, a1_ref, z_ref, out_ref, *, inv_n, blk):
    i = pl.program_id(0)
    zfull = z_ref[...]
    contrib = jnp.float32(0.0)
    for k, a_ref in enumerate((a0_ref, a1_ref)):
        a = a_ref[...]
        zi = z_ref[pl.ds((i + k * 8) * blk, blk), :]
        y = jnp.dot(a, zfull, preferred_element_type=jnp.float32)
        d = jnp.sum(a, axis=1)
        s = jnp.sum(zi * zi, axis=1)
        contrib += jnp.sum(d * s) - jnp.sum(zi * y)
    contrib = jnp.reshape(contrib * inv_n, (1, 1))

    @pl.when(i == 0)
    def _init():
        out_ref[...] = contrib

    @pl.when(i != 0)
    def _acc():
        out_ref[...] += contrib


@jax.jit
def kernel(z, coords, precomputed_adj):
    del coords
    n, dim = z.shape
    blk = 256
    grid = (n // (2 * blk),)
    out = pl.pallas_call(
        functools.partial(_smoothness_body, inv_n=1.0 / n, blk=blk),
        grid=grid,
        in_specs=[
            pl.BlockSpec((blk, n), lambda i: (i, 0)),
            pl.BlockSpec((blk, n), lambda i: (8 + i, 0)),
            pl.BlockSpec((n, dim), lambda i: (0, 0)),
        ],
        out_specs=pl.BlockSpec((1, 1), lambda i: (0, 0)),
        out_shape=jax.ShapeDtypeStruct((1, 1), jnp.float32),
    )(precomputed_adj, precomputed_adj, z)
    return out[0, 0]


# speedup vs baseline: 1.0543x; 1.0187x over previous
import functools

import jax
import jax.numpy as jnp
from jax.experimental import pallas as pl


def _smoothness_body(a0_ref, a1_ref, z_ref, out_ref, *, inv_n, blk):
    i = pl.program_id(0)
    zfull = z_ref[...]
    contrib = jnp.float32(0.0)
    for k, a_ref in enumerate((a0_ref, a1_ref)):
        a = a_ref[...]
        zi = z_ref[pl.ds((2 * i + k) * blk, blk), :]
        y = jnp.dot(a, zfull, preferred_element_type=jnp.float32)
        d = jnp.sum(a, axis=1)
        s = jnp.sum(zi * zi, axis=1)
        contrib += jnp.sum(d * s) - jnp.sum(zi * y)
    contrib = jnp.reshape(contrib * inv_n, (1, 1))

    @pl.when(i == 0)
    def _init():
        out_ref[...] = contrib

    @pl.when(i != 0)
    def _acc():
        out_ref[...] += contrib


@jax.jit
def kernel(z, coords, precomputed_adj):
    del coords
    n, dim = z.shape
    blk = 256
    grid = (n // (2 * blk),)
    out = pl.pallas_call(
        functools.partial(_smoothness_body, inv_n=1.0 / n, blk=blk),
        grid=grid,
        in_specs=[
            pl.BlockSpec((blk, n), lambda i: (2 * i, 0)),
            pl.BlockSpec((blk, n), lambda i: (2 * i + 1, 0)),
            pl.BlockSpec((n, dim), lambda i: (0, 0)),
        ],
        out_specs=pl.BlockSpec((1, 1), lambda i: (0, 0)),
        out_shape=jax.ShapeDtypeStruct((1, 1), jnp.float32),
    )(precomputed_adj, precomputed_adj, z)
    return out[0, 0]
